# Initial kernel scaffold; baseline (speedup 1.0000x reference)
import jax
import jax.numpy as jnp
from jax.experimental import pallas as pl


def kernel(x, edge_index, W, b):
    raise NotImplementedError("write your pallas kernel here")


# speedup vs baseline: 5.7259x; 5.7259x over previous
import functools

import jax
import jax.numpy as jnp
from jax import lax
from jax.experimental import pallas as pl
from jax.experimental.pallas import tpu as pltpu
from jax.experimental.pallas import tpu_sc as plsc

N_NODES = 10000
N_PAD = 10240
N_EDGES = 320000
D = 128

NW = 32
EPW = N_EDGES // NW
CHUNK = 80
NCHUNK = EPW // CHUNK
ROWS_PER_TILE = N_PAD // 16


def _tables_body(x_ref, ws_ref, wd_ref, b_ref, a_ref, c_ref):
    xb = x_ref[...]
    a_ref[...] = jnp.dot(xb, ws_ref[...], preferred_element_type=jnp.float32)
    c_ref[...] = jnp.dot(xb, wd_ref[...], preferred_element_type=jnp.float32) + b_ref[...]


def _node_tables(x, ws, wd, b2):
    blk = N_NODES // 10
    return pl.pallas_call(
        _tables_body,
        grid=(10,),
        in_specs=[
            pl.BlockSpec((blk, D), lambda i: (i, 0)),
            pl.BlockSpec((D, D), lambda i: (0, 0)),
            pl.BlockSpec((D, D), lambda i: (0, 0)),
            pl.BlockSpec((1, D), lambda i: (0, 0)),
        ],
        out_specs=[
            pl.BlockSpec((blk, D), lambda i: (i, 0)),
            pl.BlockSpec((blk, D), lambda i: (i, 0)),
        ],
        out_shape=[jax.ShapeDtypeStruct((N_NODES, D), jnp.float32)] * 2,
    )(x, ws, wd, b2)


def _combine_body(p_ref, o_ref):
    o_ref[...] = p_ref[0] + p_ref[1]


def _combine(partial):
    blk = N_PAD // 10
    return pl.pallas_call(
        _combine_body,
        grid=(10,),
        in_specs=[pl.BlockSpec((2, blk, D), lambda i: (0, i, 0))],
        out_specs=pl.BlockSpec((blk, D), lambda i: (i, 0)),
        out_shape=jax.ShapeDtypeStruct((N_PAD, D), jnp.float32),
    )(partial)


_SC_MESH = plsc.VectorSubcoreMesh(core_axis_name="c", subcore_axis_name="s")


@functools.partial(
    pl.kernel,
    out_type=jax.ShapeDtypeStruct((2, N_PAD, D), jnp.float32),
    mesh=_SC_MESH,
    scratch_types=[
        pltpu.VMEM((CHUNK,), jnp.int32),
        pltpu.VMEM((CHUNK,), jnp.int32),
        pltpu.VMEM((CHUNK, D), jnp.float32),
        pltpu.VMEM((CHUNK, D), jnp.float32),
        pltpu.VMEM_SHARED((N_PAD, D), jnp.float32),
        pltpu.SemaphoreType.DMA,
        pltpu.SemaphoreType.DMA,
    ],
)
def _sc_edges(a_hbm, c_hbm, src_hbm, dst_hbm, out_hbm,
              sidx, didx, abuf, cbuf, acc, sem_a, sem_c):
    cid = lax.axis_index("c")
    sid = lax.axis_index("s")
    wid = sid * 2 + cid
    ebase = wid * EPW

    zero = jnp.zeros((16,), jnp.float32)

    def zrow(r, carry):
        for cc in range(8):
            cbuf[r, pl.ds(cc * 16, 16)] = zero
        return carry

    lax.fori_loop(0, CHUNK, zrow, 0)
    base = sid * ROWS_PER_TILE
    for k in range(ROWS_PER_TILE // CHUNK):
        pltpu.sync_copy(cbuf, acc.at[pl.ds(base + k * CHUNK, CHUNK)])
    plsc.subcore_barrier()

    def chunk(j, carry):
        off = ebase + j * CHUNK
        pltpu.sync_copy(src_hbm.at[pl.ds(off, CHUNK)], sidx)
        pltpu.sync_copy(dst_hbm.at[pl.ds(off, CHUNK)], didx)
        ga = pltpu.async_copy(a_hbm.at[sidx], abuf, sem_a)
        gc = pltpu.async_copy(c_hbm.at[didx], cbuf, sem_c)
        ga.wait()
        gc.wait()

        def row(r, rc):
            for cc in range(8):
                s = pl.ds(cc * 16, 16)
                abuf[r, s] = jnp.maximum(abuf[r, s] + cbuf[r, s], 0.0)
            return rc

        lax.fori_loop(0, CHUNK, row, 0)
        pltpu.sync_copy(abuf, acc.at[didx], add=True)
        return carry

    lax.fori_loop(0, NCHUNK, chunk, 0)
    plsc.subcore_barrier()

    pltpu.sync_copy(acc.at[pl.ds(base, ROWS_PER_TILE)],
                    out_hbm.at[cid, pl.ds(base, ROWS_PER_TILE)])


def kernel(x, edge_index, W, b):
    ei = edge_index.astype(jnp.int32)
    src = ei[0]
    dst = ei[1]
    ws = W[:D]
    wd = W[D:]
    a_tab, c_tab = _node_tables(x, ws, wd, b.reshape(1, D))
    partial = _sc_edges(a_tab, c_tab, src, dst)
    return _combine(partial)[:N_NODES]


# speedup vs baseline: 11.4019x; 1.9913x over previous
import functools

import jax
import jax.numpy as jnp
from jax import lax
from jax.experimental import pallas as pl
from jax.experimental.pallas import tpu as pltpu
from jax.experimental.pallas import tpu_sc as plsc

N_NODES = 10000
N_PAD = 10240
N_EDGES = 320000
D = 128

NW = 32
EPW = N_EDGES // NW
CHUNK = 80
NCHUNK = EPW // CHUNK
ROWS_PER_TILE = N_PAD // 16


def _tables_body(x_ref, ws_ref, wd_ref, b_ref, a_ref, c_ref):
    xb = x_ref[...]
    a_ref[...] = jnp.dot(xb, ws_ref[...], preferred_element_type=jnp.float32)
    c_ref[...] = jnp.dot(xb, wd_ref[...], preferred_element_type=jnp.float32) + b_ref[...]


def _node_tables(x, ws, wd, b2):
    blk = N_NODES // 10
    return pl.pallas_call(
        _tables_body,
        grid=(10,),
        in_specs=[
            pl.BlockSpec((blk, D), lambda i: (i, 0)),
            pl.BlockSpec((D, D), lambda i: (0, 0)),
            pl.BlockSpec((D, D), lambda i: (0, 0)),
            pl.BlockSpec((1, D), lambda i: (0, 0)),
        ],
        out_specs=[
            pl.BlockSpec((blk, D), lambda i: (i, 0)),
            pl.BlockSpec((blk, D), lambda i: (i, 0)),
        ],
        out_shape=[jax.ShapeDtypeStruct((N_NODES, D), jnp.float32)] * 2,
    )(x, ws, wd, b2)


def _combine_body(p_ref, o_ref):
    o_ref[...] = p_ref[0] + p_ref[1]


def _combine(partial):
    blk = N_PAD // 10
    return pl.pallas_call(
        _combine_body,
        grid=(10,),
        in_specs=[pl.BlockSpec((2, blk, D), lambda i: (0, i, 0))],
        out_specs=pl.BlockSpec((blk, D), lambda i: (i, 0)),
        out_shape=jax.ShapeDtypeStruct((N_PAD, D), jnp.float32),
    )(partial)


_SC_MESH = plsc.VectorSubcoreMesh(core_axis_name="c", subcore_axis_name="s")


@functools.partial(
    pl.kernel,
    out_type=jax.ShapeDtypeStruct((2, N_PAD, D), jnp.float32),
    mesh=_SC_MESH,
    scratch_types=[
        pltpu.VMEM((2, CHUNK), jnp.int32),
        pltpu.VMEM((2, CHUNK), jnp.int32),
        pltpu.VMEM((2, CHUNK), jnp.int32),
        pltpu.VMEM((2, CHUNK, D), jnp.float32),
        pltpu.VMEM((2, CHUNK, D), jnp.float32),
        pltpu.VMEM_SHARED((N_PAD, D), jnp.float32),
        pltpu.SemaphoreType.DMA,
        pltpu.SemaphoreType.DMA,
        pltpu.SemaphoreType.DMA,
        pltpu.SemaphoreType.DMA,
        pltpu.SemaphoreType.DMA,
        pltpu.SemaphoreType.DMA,
        pltpu.SemaphoreType.DMA,
        pltpu.SemaphoreType.DMA,
        pltpu.SemaphoreType.DMA,
        pltpu.SemaphoreType.DMA,
    ],
)
def _sc_edges(a_hbm, c_hbm, src_hbm, dst_hbm, out_hbm,
              sidx, didx, wdix, abuf, cbuf, acc,
              sem_ga0, sem_ga1, sem_gc0, sem_gc1,
              sem_is0, sem_is1, sem_id0, sem_id1,
              sem_w0, sem_w1):
    cid = lax.axis_index("c")
    sid = lax.axis_index("s")
    wid = sid * 2 + cid
    ebase = wid * EPW

    sem_ga = (sem_ga0, sem_ga1)
    sem_gc = (sem_gc0, sem_gc1)
    sem_is = (sem_is0, sem_is1)
    sem_id = (sem_id0, sem_id1)
    sem_w = (sem_w0, sem_w1)

    zero = jnp.zeros((16,), jnp.float32)

    def zrow(r, carry):
        for cc in range(8):
            cbuf[0, r, pl.ds(cc * 16, 16)] = zero
        return carry

    lax.fori_loop(0, CHUNK, zrow, 0)
    base = sid * ROWS_PER_TILE
    for k in range(ROWS_PER_TILE // CHUNK):
        pltpu.sync_copy(cbuf.at[0], acc.at[pl.ds(base + k * CHUNK, CHUNK)])
    plsc.subcore_barrier()

    def wait_bytes(dst_ref, sem):
        pltpu.make_async_copy(a_hbm.at[pl.ds(0, dst_ref.shape[0])], dst_ref, sem).wait()

    def wait_idx_bytes(dst_ref, sem):
        pltpu.make_async_copy(src_hbm.at[pl.ds(0, CHUNK)], dst_ref, sem).wait()

    def issue_idx(j, p):
        off = ebase + j * CHUNK
        pltpu.async_copy(src_hbm.at[pl.ds(off, CHUNK)], sidx.at[p], sem_is[p])
        pltpu.async_copy(dst_hbm.at[pl.ds(off, CHUNK)], didx.at[p], sem_id[p])

    def issue_gathers(p):
        pltpu.async_copy(a_hbm.at[sidx.at[p]], abuf.at[p], sem_ga[p])
        pltpu.async_copy(c_hbm.at[didx.at[p]], cbuf.at[p], sem_gc[p])

    pltpu.sync_copy(src_hbm.at[pl.ds(ebase, CHUNK)], sidx.at[0])
    pltpu.sync_copy(dst_hbm.at[pl.ds(ebase, CHUNK)], didx.at[0])
    pltpu.sync_copy(src_hbm.at[pl.ds(ebase + CHUNK, CHUNK)], sidx.at[1])
    pltpu.sync_copy(dst_hbm.at[pl.ds(ebase + CHUNK, CHUNK)], didx.at[1])
    issue_gathers(0)

    def do_chunk(j, p, tail):
        q = 1 - p

        @pl.when(j >= 1)
        def _():
            wait_bytes(abuf.at[q], sem_w[q])

        @pl.when(jnp.logical_and(j >= 1, j + 1 < NCHUNK))
        def _():
            wait_idx_bytes(sidx.at[q], sem_is[q])
            wait_idx_bytes(didx.at[q], sem_id[q])

        @pl.when(j + 1 < NCHUNK)
        def _():
            issue_gathers(q)

        wait_bytes(abuf.at[p], sem_ga[p])
        wait_bytes(cbuf.at[p], sem_gc[p])

        for g in range(CHUNK // 16):
            wdix[p, pl.ds(g * 16, 16)] = didx[p, pl.ds(g * 16, 16)]

        @pl.when(j + 2 < NCHUNK)
        def _():
            issue_idx(j + 2, p)

        def row(r, rc):
            for cc in range(8):
                s = pl.ds(cc * 16, 16)
                abuf[p, r, s] = jnp.maximum(abuf[p, r, s] + cbuf[p, r, s], 0.0)
            return rc

        lax.fori_loop(0, CHUNK, row, 0)
        if tail:
            pltpu.sync_copy(abuf.at[p], acc.at[wdix.at[p]], add=True)
        else:
            pltpu.async_copy(abuf.at[p], acc.at[wdix.at[p]], sem_w[p], add=True)

    def pair(i, carry):
        do_chunk(2 * i, 0, False)
        do_chunk(2 * i + 1, 1, False)
        return carry

    lax.fori_loop(0, (NCHUNK - 1) // 2, pair, 0)
    do_chunk(jnp.int32(NCHUNK - 1), 0, True)
    plsc.subcore_barrier()

    pltpu.sync_copy(acc.at[pl.ds(base, ROWS_PER_TILE)],
                    out_hbm.at[cid, pl.ds(base, ROWS_PER_TILE)])


def kernel(x, edge_index, W, b):
    ei = edge_index.astype(jnp.int32)
    src = ei[0]
    dst = ei[1]
    ws = W[:D]
    wd = W[D:]
    a_tab, c_tab = _node_tables(x, ws, wd, b.reshape(1, D))
    partial = _sc_edges(a_tab, c_tab, src, dst)
    return _combine(partial)[:N_NODES]


# speedup vs baseline: 12.8566x; 1.1276x over previous
import functools

import jax
import jax.numpy as jnp
from jax import lax
from jax.experimental import pallas as pl
from jax.experimental.pallas import tpu as pltpu
from jax.experimental.pallas import tpu_sc as plsc

N_NODES = 10000
N_PAD = 10240
N_EDGES = 320000
D = 128
DP = D // 2

NW = 32
EPW = N_EDGES // NW
CHUNK = 80
NCHUNK = EPW // CHUNK
ROWS_PER_TILE = N_PAD // 16


def _pack_halves(v):
    u = lax.bitcast_convert_type(v.astype(jnp.bfloat16), jnp.uint16)
    lo = u[:, :DP].astype(jnp.uint32)
    hi = u[:, DP:].astype(jnp.uint32)
    return lax.bitcast_convert_type(lo | (hi << 16), jnp.float32)


def _tables_body(x_ref, ws_ref, wd_ref, b_ref, a_ref, c_ref):
    xb = x_ref[...]
    a = jnp.dot(xb, ws_ref[...], preferred_element_type=jnp.float32)
    c = jnp.dot(xb, wd_ref[...], preferred_element_type=jnp.float32) + b_ref[...]
    a_ref[...] = _pack_halves(a)
    c_ref[...] = _pack_halves(c)


def _node_tables(x, ws, wd, b2):
    blk = N_NODES // 10
    return pl.pallas_call(
        _tables_body,
        grid=(10,),
        in_specs=[
            pl.BlockSpec((blk, D), lambda i: (i, 0)),
            pl.BlockSpec((D, D), lambda i: (0, 0)),
            pl.BlockSpec((D, D), lambda i: (0, 0)),
            pl.BlockSpec((1, D), lambda i: (0, 0)),
        ],
        out_specs=[
            pl.BlockSpec((blk, DP), lambda i: (i, 0)),
            pl.BlockSpec((blk, DP), lambda i: (i, 0)),
        ],
        out_shape=[jax.ShapeDtypeStruct((N_NODES, DP), jnp.float32)] * 2,
    )(x, ws, wd, b2)


def _combine_body(pe_ref, po_ref, o_ref):
    e = pe_ref[0] + pe_ref[1]
    o = po_ref[0] + po_ref[1]
    o_ref[...] = jnp.concatenate([e, o], axis=-1)


def _combine(pe, po):
    blk = N_NODES // 10
    return pl.pallas_call(
        _combine_body,
        grid=(10,),
        in_specs=[pl.BlockSpec((2, blk, DP), lambda i: (0, i, 0)),
                  pl.BlockSpec((2, blk, DP), lambda i: (0, i, 0))],
        out_specs=pl.BlockSpec((blk, D), lambda i: (i, 0)),
        out_shape=jax.ShapeDtypeStruct((N_NODES, D), jnp.float32),
    )(pe, po)


_SC_MESH = plsc.VectorSubcoreMesh(core_axis_name="c", subcore_axis_name="s")


@functools.partial(
    pl.kernel,
    out_type=[jax.ShapeDtypeStruct((2, N_PAD, DP), jnp.float32)] * 2,
    mesh=_SC_MESH,
    compiler_params=pltpu.CompilerParams(use_tc_tiling_on_sc=False),
    scratch_types=[
        pltpu.VMEM((2, CHUNK), jnp.int32),
        pltpu.VMEM((2, CHUNK), jnp.int32),
        pltpu.VMEM((2, CHUNK), jnp.int32),
        pltpu.VMEM((2, CHUNK, DP), jnp.float32),
        pltpu.VMEM((2, CHUNK, DP), jnp.float32),
        pltpu.VMEM_SHARED((N_PAD, DP), jnp.float32),
        pltpu.VMEM_SHARED((N_PAD, DP), jnp.float32),
        pltpu.SemaphoreType.DMA,
        pltpu.SemaphoreType.DMA,
        pltpu.SemaphoreType.DMA,
        pltpu.SemaphoreType.DMA,
        pltpu.SemaphoreType.DMA,
        pltpu.SemaphoreType.DMA,
        pltpu.SemaphoreType.DMA,
        pltpu.SemaphoreType.DMA,
        pltpu.SemaphoreType.DMA,
        pltpu.SemaphoreType.DMA,
    ],
)
def _sc_edges(a_hbm, c_hbm, src_hbm, dst_hbm, oute_hbm, outo_hbm,
              sidx, didx, wdix, abuf, cbuf, acc_e, acc_o,
              sem_ga0, sem_ga1, sem_gc0, sem_gc1,
              sem_is0, sem_is1, sem_id0, sem_id1,
              sem_w0, sem_w1):
    cid = lax.axis_index("c")
    sid = lax.axis_index("s")
    wid = sid * 2 + cid
    ebase = wid * EPW

    sem_ga = (sem_ga0, sem_ga1)
    sem_gc = (sem_gc0, sem_gc1)
    sem_is = (sem_is0, sem_is1)
    sem_id = (sem_id0, sem_id1)
    sem_w = (sem_w0, sem_w1)

    zero = jnp.zeros((16,), jnp.float32)

    def zrow(r, carry):
        for cc in range(DP // 16):
            cbuf[0, r, pl.ds(cc * 16, 16)] = zero
        return carry

    lax.fori_loop(0, CHUNK, zrow, 0)
    base = sid * ROWS_PER_TILE
    for k in range(ROWS_PER_TILE // CHUNK):
        pltpu.sync_copy(cbuf.at[0], acc_e.at[pl.ds(base + k * CHUNK, CHUNK)])
        pltpu.sync_copy(cbuf.at[0], acc_o.at[pl.ds(base + k * CHUNK, CHUNK)])
    plsc.subcore_barrier()

    def wait_bytes(dst_ref, sem):
        pltpu.make_async_copy(a_hbm.at[pl.ds(0, dst_ref.shape[0])], dst_ref, sem).wait()

    def wait_idx_bytes(dst_ref, sem):
        pltpu.make_async_copy(src_hbm.at[pl.ds(0, CHUNK)], dst_ref, sem).wait()

    def issue_idx(j, p):
        off = ebase + j * CHUNK
        pltpu.async_copy(src_hbm.at[pl.ds(off, CHUNK)], sidx.at[p], sem_is[p])
        pltpu.async_copy(dst_hbm.at[pl.ds(off, CHUNK)], didx.at[p], sem_id[p])

    def issue_gathers(p):
        pltpu.async_copy(a_hbm.at[sidx.at[p]], abuf.at[p], sem_ga[p])
        pltpu.async_copy(c_hbm.at[didx.at[p]], cbuf.at[p], sem_gc[p])

    pltpu.sync_copy(src_hbm.at[pl.ds(ebase, CHUNK)], sidx.at[0])
    pltpu.sync_copy(dst_hbm.at[pl.ds(ebase, CHUNK)], didx.at[0])
    pltpu.sync_copy(src_hbm.at[pl.ds(ebase + CHUNK, CHUNK)], sidx.at[1])
    pltpu.sync_copy(dst_hbm.at[pl.ds(ebase + CHUNK, CHUNK)], didx.at[1])
    issue_gathers(0)

    def do_chunk(j, p, tail):
        q = 1 - p

        @pl.when(j >= 1)
        def _():
            wait_bytes(abuf.at[q], sem_w[q])
            wait_bytes(cbuf.at[q], sem_w[q])

        @pl.when(jnp.logical_and(j >= 1, j + 1 < NCHUNK))
        def _():
            wait_idx_bytes(sidx.at[q], sem_is[q])
            wait_idx_bytes(didx.at[q], sem_id[q])

        @pl.when(j + 1 < NCHUNK)
        def _():
            issue_gathers(q)

        wait_bytes(abuf.at[p], sem_ga[p])
        wait_bytes(cbuf.at[p], sem_gc[p])

        for g in range(CHUNK // 16):
            wdix[p, pl.ds(g * 16, 16)] = didx[p, pl.ds(g * 16, 16)]

        @pl.when(j + 2 < NCHUNK)
        def _():
            issue_idx(j + 2, p)

        def row(r, rc):
            mask_hi = jnp.int32(-65536)
            for cc in range(DP // 16):
                s = pl.ds(cc * 16, 16)
                wa = lax.bitcast_convert_type(abuf[p, r, s], jnp.int32)
                wc = lax.bitcast_convert_type(cbuf[p, r, s], jnp.int32)
                alo = lax.bitcast_convert_type(wa << 16, jnp.float32)
                clo = lax.bitcast_convert_type(wc << 16, jnp.float32)
                ahi = lax.bitcast_convert_type(wa & mask_hi, jnp.float32)
                chi = lax.bitcast_convert_type(wc & mask_hi, jnp.float32)
                abuf[p, r, s] = jnp.maximum(alo + clo, 0.0)
                cbuf[p, r, s] = jnp.maximum(ahi + chi, 0.0)
            return rc

        lax.fori_loop(0, CHUNK, row, 0)
        if tail:
            pltpu.sync_copy(abuf.at[p], acc_e.at[wdix.at[p]], add=True)
            pltpu.sync_copy(cbuf.at[p], acc_o.at[wdix.at[p]], add=True)
        else:
            pltpu.async_copy(abuf.at[p], acc_e.at[wdix.at[p]], sem_w[p], add=True)
            pltpu.async_copy(cbuf.at[p], acc_o.at[wdix.at[p]], sem_w[p], add=True)

    def pair(i, carry):
        do_chunk(2 * i, 0, False)
        do_chunk(2 * i + 1, 1, False)
        return carry

    lax.fori_loop(0, (NCHUNK - 1) // 2, pair, 0)
    do_chunk(jnp.int32(NCHUNK - 1), 0, True)
    plsc.subcore_barrier()

    pltpu.sync_copy(acc_e.at[pl.ds(base, ROWS_PER_TILE)],
                    oute_hbm.at[cid, pl.ds(base, ROWS_PER_TILE)])
    pltpu.sync_copy(acc_o.at[pl.ds(base, ROWS_PER_TILE)],
                    outo_hbm.at[cid, pl.ds(base, ROWS_PER_TILE)])


def kernel(x, edge_index, W, b):
    ei = edge_index.astype(jnp.int32)
    src = ei[0]
    dst = ei[1]
    ws = W[:D]
    wd = W[D:]
    a_tab, c_tab = _node_tables(x, ws, wd, b.reshape(1, D))
    pe, po = _sc_edges(a_tab, c_tab, src, dst)
    return _combine(pe, po)
